# baseline (speedup 1.0000x reference)
import jax
import jax.numpy as jnp
from jax.experimental import pallas as pl
from jax.experimental.pallas import tpu as pltpu
from jax.experimental.pallas import tpu_sc as plsc

_WINDOW = 128

_VOCAB = 1000000
_DIM = 64
_N_IDX = 819200

_A_BLK = 4096
_A_GRID = 123
_HALF = _A_BLK * _A_GRID
_C_BLK = 6400
_C_GRID = _N_IDX // _C_BLK

_N_WINDOWS = _N_IDX // _WINDOW
_WPC = _C_BLK // _WINDOW


def _relayout_table(t_view):
    def body(x_lo, x_hi, o_ref):
        x = jnp.concatenate([x_lo[...], x_hi[...]], axis=0)
        o_ref[...] = jnp.swapaxes(x, 0, 1)

    return pl.pallas_call(
        body,
        grid=(_A_GRID,),
        in_specs=[
            pl.BlockSpec((_DIM, _A_BLK), lambda c: (0, c)),
            pl.BlockSpec(
                (_DIM, _A_BLK),
                lambda c: (0, jnp.minimum(c + _A_GRID, _VOCAB // _A_BLK)),
            ),
        ],
        out_specs=pl.BlockSpec((_A_BLK, 128), lambda c: (c, 0)),
        out_shape=jax.ShapeDtypeStruct((_HALF, 128), jnp.float32),
        compiler_params=pltpu.CompilerParams(
            dimension_semantics=("parallel",)
        ),
    )(t_view, t_view)


def _remap(v):
    t2 = v + v
    return jnp.where(v < _HALF, t2, t2 - (2 * _HALF - 1))


_N_CHUNKS = 2
_CW = _N_WINDOWS // _N_CHUNKS
_CN = _N_IDX // _N_CHUNKS


def _sc_gather(table_lin, idx_chunk):
    mesh = plsc.VectorSubcoreMesh(core_axis_name="c", subcore_axis_name="s")

    @pl.kernel(
        out_type=jax.ShapeDtypeStruct((_CN, _DIM), jnp.float32),
        mesh=mesh,
        scratch_types=[pltpu.VMEM((_WINDOW,), jnp.int32)],
        compiler_params=pltpu.CompilerParams(
            use_tc_tiling_on_sc=False, needs_layout_passes=False
        ),
    )
    def embed_gather(x_hbm, i_hbm, o_hbm, scr):
        lane = jax.lax.iota(jnp.int32, 16)

        def body(i_a, i_b, o_vmem):
            for n in range(4):
                a = _remap(i_a[0, pl.ds(16 * n, 16)])
                plsc.store_scatter(scr, [2 * (lane + 16 * n)], a)
                b = _remap(i_b[0, pl.ds(16 * n, 16)])
                plsc.store_scatter(scr, [2 * (lane + 16 * n) + 1], b)
            pltpu.sync_copy(x_hbm.at[scr], o_vmem)

        def amap(i):
            u = 100 * (i // _WPC) + i % _WPC
            return (u // 2, u % 2)

        def bmap(i):
            u = 100 * (i // _WPC) + i % _WPC + _WPC
            return (u // 2, u % 2)

        pltpu.emit_pipeline(
            body,
            grid=(_CW,),
            in_specs=[
                pl.BlockSpec((1, 64), index_map=amap),
                pl.BlockSpec((1, 64), index_map=bmap),
            ],
            out_specs=[
                pl.BlockSpec((_WINDOW, _DIM), index_map=lambda i: (i, 0)),
            ],
            core_axis_name=("c", "s"),
            dimension_semantics=(pltpu.PARALLEL,),
        )(i_hbm, i_hbm, o_hbm)

    return embed_gather(table_lin, idx_chunk)


def _relayout_out(out_view, k, prev):
    def body(x_ref, o_ref):
        y = jnp.swapaxes(x_ref[...], 0, 1)
        o_ref[...] = jnp.concatenate([y[0:64, :], y[64:128, :]], axis=1)

    blk0 = k * (_CN // _C_BLK)
    operands = [out_view]
    in_specs = [pl.BlockSpec((_C_BLK // 2, 128), lambda r: (r, 0))]
    aliases = {}
    if prev is not None:
        operands.append(prev)
        in_specs.append(pl.BlockSpec(memory_space=pl.ANY))
        aliases = {1: 0}

    def body_wrap(*refs):
        body(refs[0], refs[-1])

    return pl.pallas_call(
        body_wrap,
        grid=(_CN // _C_BLK,),
        in_specs=in_specs,
        out_specs=pl.BlockSpec((64, _C_BLK), lambda r: (0, r + blk0)),
        out_shape=jax.ShapeDtypeStruct((64, _N_IDX), jnp.float32),
        input_output_aliases=aliases,
        compiler_params=pltpu.CompilerParams(
            dimension_semantics=("parallel",)
        ),
    )(*operands)


def kernel(token_id, emb_norm):
    idx_nat = token_id.reshape(_N_WINDOWS, _WINDOW).astype(jnp.int32)
    table_folded = _relayout_table(emb_norm.T)
    table_lin = table_folded.reshape(2 * _HALF, _DIM)
    out_t = None
    for k in range(_N_CHUNKS):
        idx_chunk = jax.lax.slice_in_dim(idx_nat, k * _CW, (k + 1) * _CW)
        gathered = _sc_gather(table_lin, idx_chunk)
        out_t = _relayout_out(gathered.reshape(_CN // 2, 128), k, out_t)
    return out_t.T


# speedup vs baseline: 1.0211x; 1.0211x over previous
import jax
import jax.numpy as jnp
from jax.experimental import pallas as pl
from jax.experimental.pallas import tpu as pltpu
from jax.experimental.pallas import tpu_sc as plsc

_WINDOW = 128

_VOCAB = 1000000
_DIM = 64
_N_IDX = 819200

_A_BLK = 4096
_A_GRID = 123
_HALF = _A_BLK * _A_GRID
_C_BLK = 6400
_C_GRID = _N_IDX // _C_BLK

_N_WINDOWS = _N_IDX // _WINDOW
_WPC = _C_BLK // _WINDOW


def _relayout_table(t_view):
    def body(x_lo, x_hi, o_ref):
        x = jnp.concatenate([x_lo[...], x_hi[...]], axis=0)
        o_ref[...] = jnp.swapaxes(x, 0, 1)

    return pl.pallas_call(
        body,
        grid=(_A_GRID,),
        in_specs=[
            pl.BlockSpec((_DIM, _A_BLK), lambda c: (0, c)),
            pl.BlockSpec(
                (_DIM, _A_BLK),
                lambda c: (0, jnp.minimum(c + _A_GRID, _VOCAB // _A_BLK)),
            ),
        ],
        out_specs=pl.BlockSpec((_A_BLK, 128), lambda c: (c, 0)),
        out_shape=jax.ShapeDtypeStruct((_HALF, 128), jnp.float32),
        compiler_params=pltpu.CompilerParams(
            dimension_semantics=("parallel",)
        ),
    )(t_view, t_view)


def _remap(v):
    t2 = v + v
    return jnp.where(v < _HALF, t2, t2 - (2 * _HALF - 1))


_N_CHUNKS = 4
_CW = _N_WINDOWS // _N_CHUNKS
_CN = _N_IDX // _N_CHUNKS


def _sc_gather(table_lin, idx_nat, k):
    base = k * _CW
    mesh = plsc.VectorSubcoreMesh(core_axis_name="c", subcore_axis_name="s")

    @pl.kernel(
        out_type=jax.ShapeDtypeStruct((_CN, _DIM), jnp.float32),
        mesh=mesh,
        scratch_types=[pltpu.VMEM((_WINDOW,), jnp.int32)],
        compiler_params=pltpu.CompilerParams(
            use_tc_tiling_on_sc=False, needs_layout_passes=False
        ),
    )
    def embed_gather(x_hbm, i_hbm, o_hbm, scr):
        lane = jax.lax.iota(jnp.int32, 16)

        def body(i_a, i_b, o_vmem):
            for n in range(4):
                a = _remap(i_a[0, pl.ds(16 * n, 16)])
                plsc.store_scatter(scr, [2 * (lane + 16 * n)], a)
                b = _remap(i_b[0, pl.ds(16 * n, 16)])
                plsc.store_scatter(scr, [2 * (lane + 16 * n) + 1], b)
            pltpu.sync_copy(x_hbm.at[scr], o_vmem)

        def amap(i):
            g = i + base
            u = 100 * (g // _WPC) + g % _WPC
            return (u // 2, u % 2)

        def bmap(i):
            g = i + base
            u = 100 * (g // _WPC) + g % _WPC + _WPC
            return (u // 2, u % 2)

        pltpu.emit_pipeline(
            body,
            grid=(_CW,),
            in_specs=[
                pl.BlockSpec((1, 64), index_map=amap),
                pl.BlockSpec((1, 64), index_map=bmap),
            ],
            out_specs=[
                pl.BlockSpec((_WINDOW, _DIM), index_map=lambda i: (i, 0)),
            ],
            core_axis_name=("c", "s"),
            dimension_semantics=(pltpu.PARALLEL,),
        )(i_hbm, i_hbm, o_hbm)

    return embed_gather(table_lin, idx_nat)


def _relayout_out(out_view, k, prev):
    def body(x_ref, o_ref):
        y = jnp.swapaxes(x_ref[...], 0, 1)
        o_ref[...] = jnp.concatenate([y[0:64, :], y[64:128, :]], axis=1)

    blk0 = k * (_CN // _C_BLK)
    operands = [out_view]
    in_specs = [pl.BlockSpec((_C_BLK // 2, 128), lambda r: (r, 0))]
    aliases = {}
    if prev is not None:
        operands.append(prev)
        in_specs.append(pl.BlockSpec(memory_space=pl.ANY))
        aliases = {1: 0}

    def body_wrap(*refs):
        body(refs[0], refs[-1])

    return pl.pallas_call(
        body_wrap,
        grid=(_CN // _C_BLK,),
        in_specs=in_specs,
        out_specs=pl.BlockSpec((64, _C_BLK), lambda r: (0, r + blk0)),
        out_shape=jax.ShapeDtypeStruct((64, _N_IDX), jnp.float32),
        input_output_aliases=aliases,
        compiler_params=pltpu.CompilerParams(
            dimension_semantics=("parallel",)
        ),
    )(*operands)


def kernel(token_id, emb_norm):
    idx_nat = token_id.reshape(_N_WINDOWS, _WINDOW).astype(jnp.int32)
    table_folded = _relayout_table(emb_norm.T)
    table_lin = table_folded.reshape(2 * _HALF, _DIM)
    out_t = None
    for k in range(_N_CHUNKS):
        gathered = _sc_gather(table_lin, idx_nat, k)
        out_t = _relayout_out(gathered.reshape(_CN // 2, 128), k, out_t)
    return out_t.T


# speedup vs baseline: 1.0714x; 1.0493x over previous
import jax
import jax.numpy as jnp
from jax.experimental import pallas as pl
from jax.experimental.pallas import tpu as pltpu
from jax.experimental.pallas import tpu_sc as plsc

_WINDOW = 128

_VOCAB = 1000000
_DIM = 64
_N_IDX = 819200

_A_BLK = 8192
_A_GRID = 62
_HALF = _A_BLK * _A_GRID
_C_BLK = 6400
_C_GRID = _N_IDX // _C_BLK

_N_WINDOWS = _N_IDX // _WINDOW
_WPC = _C_BLK // _WINDOW


def _relayout_table(t_view):
    def body(x_lo, x_hi, o_ref):
        x = jnp.concatenate([x_lo[...], x_hi[...]], axis=0)
        o_ref[...] = jnp.swapaxes(x, 0, 1)

    return pl.pallas_call(
        body,
        grid=(_A_GRID,),
        in_specs=[
            pl.BlockSpec((_DIM, _A_BLK), lambda c: (0, c)),
            pl.BlockSpec(
                (_DIM, _A_BLK),
                lambda c: (0, jnp.minimum(c + _A_GRID, _VOCAB // _A_BLK)),
            ),
        ],
        out_specs=pl.BlockSpec((_A_BLK, 128), lambda c: (c, 0)),
        out_shape=jax.ShapeDtypeStruct((_HALF, 128), jnp.float32),
        compiler_params=pltpu.CompilerParams(
            dimension_semantics=("parallel",)
        ),
    )(t_view, t_view)


def _remap(v):
    t2 = v + v
    return jnp.where(v < _HALF, t2, t2 - (2 * _HALF - 1))


_N_CHUNKS = 4
_CW = _N_WINDOWS // _N_CHUNKS
_CN = _N_IDX // _N_CHUNKS


def _sc_gather(table_lin, idx_nat, k):
    base = k * _CW
    mesh = plsc.VectorSubcoreMesh(core_axis_name="c", subcore_axis_name="s")

    @pl.kernel(
        out_type=jax.ShapeDtypeStruct((_CN, _DIM), jnp.float32),
        mesh=mesh,
        scratch_types=[pltpu.VMEM((_WINDOW,), jnp.int32)],
        compiler_params=pltpu.CompilerParams(
            use_tc_tiling_on_sc=False, needs_layout_passes=False
        ),
    )
    def embed_gather(x_hbm, i_hbm, o_hbm, scr):
        lane = jax.lax.iota(jnp.int32, 16)

        def body(i_a, i_b, o_vmem):
            for n in range(4):
                a = _remap(i_a[0, pl.ds(16 * n, 16)])
                plsc.store_scatter(scr, [2 * (lane + 16 * n)], a)
                b = _remap(i_b[0, pl.ds(16 * n, 16)])
                plsc.store_scatter(scr, [2 * (lane + 16 * n) + 1], b)
            pltpu.sync_copy(x_hbm.at[scr], o_vmem)

        def amap(i):
            g = i + base
            u = 100 * (g // _WPC) + g % _WPC
            return (u // 2, u % 2)

        def bmap(i):
            g = i + base
            u = 100 * (g // _WPC) + g % _WPC + _WPC
            return (u // 2, u % 2)

        pltpu.emit_pipeline(
            body,
            grid=(_CW,),
            in_specs=[
                pl.BlockSpec((1, 64), index_map=amap),
                pl.BlockSpec((1, 64), index_map=bmap),
            ],
            out_specs=[
                pl.BlockSpec((_WINDOW, _DIM), index_map=lambda i: (i, 0)),
            ],
            core_axis_name=("c", "s"),
            dimension_semantics=(pltpu.PARALLEL,),
        )(i_hbm, i_hbm, o_hbm)

    return embed_gather(table_lin, idx_nat)


def _relayout_out(out_view, k, prev):
    def body(x_ref, o_ref):
        y = jnp.swapaxes(x_ref[...], 0, 1)
        o_ref[...] = jnp.concatenate([y[0:64, :], y[64:128, :]], axis=1)

    blk0 = k * (_CN // _C_BLK)
    operands = [out_view]
    in_specs = [pl.BlockSpec((_C_BLK // 2, 128), lambda r: (r, 0))]
    aliases = {}
    if prev is not None:
        operands.append(prev)
        in_specs.append(pl.BlockSpec(memory_space=pl.ANY))
        aliases = {1: 0}

    def body_wrap(*refs):
        body(refs[0], refs[-1])

    return pl.pallas_call(
        body_wrap,
        grid=(_CN // _C_BLK,),
        in_specs=in_specs,
        out_specs=pl.BlockSpec((64, _C_BLK), lambda r: (0, r + blk0)),
        out_shape=jax.ShapeDtypeStruct((64, _N_IDX), jnp.float32),
        input_output_aliases=aliases,
        compiler_params=pltpu.CompilerParams(
            dimension_semantics=("parallel",)
        ),
    )(*operands)


def kernel(token_id, emb_norm):
    idx_nat = token_id.reshape(_N_WINDOWS, _WINDOW).astype(jnp.int32)
    table_folded = _relayout_table(emb_norm.T)
    table_lin = table_folded.reshape(2 * _HALF, _DIM)
    out_t = None
    for k in range(_N_CHUNKS):
        gathered = _sc_gather(table_lin, idx_nat, k)
        out_t = _relayout_out(gathered.reshape(_CN // 2, 128), k, out_t)
    return out_t.T


# speedup vs baseline: 1.0834x; 1.0112x over previous
import jax
import jax.numpy as jnp
from jax.experimental import pallas as pl
from jax.experimental.pallas import tpu as pltpu
from jax.experimental.pallas import tpu_sc as plsc

_WINDOW = 128

_VOCAB = 1000000
_DIM = 64
_N_IDX = 819200

_A_BLK = 16384
_A_GRID = 31
_HALF = _A_BLK * _A_GRID
_C_BLK = 6400
_C_GRID = _N_IDX // _C_BLK

_N_WINDOWS = _N_IDX // _WINDOW
_WPC = _C_BLK // _WINDOW


def _relayout_table(t_view):
    def body(x_lo, x_hi, o_ref):
        x = jnp.concatenate([x_lo[...], x_hi[...]], axis=0)
        o_ref[...] = jnp.swapaxes(x, 0, 1)

    return pl.pallas_call(
        body,
        grid=(_A_GRID,),
        in_specs=[
            pl.BlockSpec((_DIM, _A_BLK), lambda c: (0, c)),
            pl.BlockSpec(
                (_DIM, _A_BLK),
                lambda c: (0, jnp.minimum(c + _A_GRID, _VOCAB // _A_BLK)),
            ),
        ],
        out_specs=pl.BlockSpec((_A_BLK, 128), lambda c: (c, 0)),
        out_shape=jax.ShapeDtypeStruct((_HALF, 128), jnp.float32),
        compiler_params=pltpu.CompilerParams(
            dimension_semantics=("parallel",)
        ),
    )(t_view, t_view)


def _remap(v):
    t2 = v + v
    return jnp.where(v < _HALF, t2, t2 - (2 * _HALF - 1))


_N_CHUNKS = 4
_CW = _N_WINDOWS // _N_CHUNKS
_CN = _N_IDX // _N_CHUNKS


def _sc_gather(table_lin, idx_nat, k):
    base = k * _CW
    mesh = plsc.VectorSubcoreMesh(core_axis_name="c", subcore_axis_name="s")

    @pl.kernel(
        out_type=jax.ShapeDtypeStruct((_CN, _DIM), jnp.float32),
        mesh=mesh,
        scratch_types=[pltpu.VMEM((_WINDOW,), jnp.int32)],
        compiler_params=pltpu.CompilerParams(
            use_tc_tiling_on_sc=False, needs_layout_passes=False
        ),
    )
    def embed_gather(x_hbm, i_hbm, o_hbm, scr):
        lane = jax.lax.iota(jnp.int32, 16)

        def body(i_a, i_b, o_vmem):
            for n in range(4):
                a = _remap(i_a[0, pl.ds(16 * n, 16)])
                plsc.store_scatter(scr, [2 * (lane + 16 * n)], a)
                b = _remap(i_b[0, pl.ds(16 * n, 16)])
                plsc.store_scatter(scr, [2 * (lane + 16 * n) + 1], b)
            pltpu.sync_copy(x_hbm.at[scr], o_vmem)

        def amap(i):
            g = i + base
            u = 100 * (g // _WPC) + g % _WPC
            return (u // 2, u % 2)

        def bmap(i):
            g = i + base
            u = 100 * (g // _WPC) + g % _WPC + _WPC
            return (u // 2, u % 2)

        pltpu.emit_pipeline(
            body,
            grid=(_CW,),
            in_specs=[
                pl.BlockSpec((1, 64), index_map=amap),
                pl.BlockSpec((1, 64), index_map=bmap),
            ],
            out_specs=[
                pl.BlockSpec((_WINDOW, _DIM), index_map=lambda i: (i, 0)),
            ],
            core_axis_name=("c", "s"),
            dimension_semantics=(pltpu.PARALLEL,),
        )(i_hbm, i_hbm, o_hbm)

    return embed_gather(table_lin, idx_nat)


def _relayout_out(out_view, k, prev):
    def body(x_ref, o_ref):
        y = jnp.swapaxes(x_ref[...], 0, 1)
        o_ref[...] = jnp.concatenate([y[0:64, :], y[64:128, :]], axis=1)

    blk0 = k * (_CN // _C_BLK)
    operands = [out_view]
    in_specs = [pl.BlockSpec((_C_BLK // 2, 128), lambda r: (r, 0))]
    aliases = {}
    if prev is not None:
        operands.append(prev)
        in_specs.append(pl.BlockSpec(memory_space=pl.ANY))
        aliases = {1: 0}

    def body_wrap(*refs):
        body(refs[0], refs[-1])

    return pl.pallas_call(
        body_wrap,
        grid=(_CN // _C_BLK,),
        in_specs=in_specs,
        out_specs=pl.BlockSpec((64, _C_BLK), lambda r: (0, r + blk0)),
        out_shape=jax.ShapeDtypeStruct((64, _N_IDX), jnp.float32),
        input_output_aliases=aliases,
        compiler_params=pltpu.CompilerParams(
            dimension_semantics=("parallel",)
        ),
    )(*operands)


def kernel(token_id, emb_norm):
    idx_nat = token_id.reshape(_N_WINDOWS, _WINDOW).astype(jnp.int32)
    table_folded = _relayout_table(emb_norm.T)
    table_lin = table_folded.reshape(2 * _HALF, _DIM)
    out_t = None
    for k in range(_N_CHUNKS):
        gathered = _sc_gather(table_lin, idx_nat, k)
        out_t = _relayout_out(gathered.reshape(_CN // 2, 128), k, out_t)
    return out_t.T


# speedup vs baseline: 1.1011x; 1.0163x over previous
import jax
import jax.numpy as jnp
from jax.experimental import pallas as pl
from jax.experimental.pallas import tpu as pltpu
from jax.experimental.pallas import tpu_sc as plsc

_WINDOW = 128

_VOCAB = 1000000
_DIM = 64
_N_IDX = 819200

_A_BLK = 16384
_A_GRID = 31
_HALF = _A_BLK * _A_GRID
_C_BLK = 6400
_C_GRID = _N_IDX // _C_BLK

_N_WINDOWS = _N_IDX // _WINDOW
_WPC = _C_BLK // _WINDOW


def _relayout_table(t_view):
    def body(x_lo, x_hi, o_ref):
        x = jnp.concatenate([x_lo[...], x_hi[...]], axis=0)
        o_ref[...] = jnp.swapaxes(x, 0, 1)

    return pl.pallas_call(
        body,
        grid=(_A_GRID,),
        in_specs=[
            pl.BlockSpec((_DIM, _A_BLK), lambda c: (0, c)),
            pl.BlockSpec(
                (_DIM, _A_BLK),
                lambda c: (0, jnp.minimum(c + _A_GRID, _VOCAB // _A_BLK)),
            ),
        ],
        out_specs=pl.BlockSpec((_A_BLK, 128), lambda c: (c, 0)),
        out_shape=jax.ShapeDtypeStruct((_HALF, 128), jnp.float32),
        compiler_params=pltpu.CompilerParams(
            dimension_semantics=("parallel",)
        ),
    )(t_view, t_view)


def _remap(v):
    t2 = v + v
    return jnp.where(v < _HALF, t2, t2 - (2 * _HALF - 1))


_CHUNKS = (2200, 1800, 1500, 900)


def _sc_gather(table_lin, idx_nat, base, n_win):
    mesh = plsc.VectorSubcoreMesh(core_axis_name="c", subcore_axis_name="s")

    @pl.kernel(
        out_type=jax.ShapeDtypeStruct((n_win * _WINDOW, _DIM), jnp.float32),
        mesh=mesh,
        scratch_types=[pltpu.VMEM((_WINDOW,), jnp.int32)],
        compiler_params=pltpu.CompilerParams(
            use_tc_tiling_on_sc=False, needs_layout_passes=False
        ),
    )
    def embed_gather(x_hbm, i_hbm, o_hbm, scr):
        lane = jax.lax.iota(jnp.int32, 16)

        def body(i_a, i_b, o_vmem):
            for n in range(4):
                a = _remap(i_a[0, pl.ds(16 * n, 16)])
                plsc.store_scatter(scr, [2 * (lane + 16 * n)], a)
                b = _remap(i_b[0, pl.ds(16 * n, 16)])
                plsc.store_scatter(scr, [2 * (lane + 16 * n) + 1], b)
            pltpu.sync_copy(x_hbm.at[scr], o_vmem)

        def amap(i):
            g = i + base
            u = 100 * (g // _WPC) + g % _WPC
            return (u // 2, u % 2)

        def bmap(i):
            g = i + base
            u = 100 * (g // _WPC) + g % _WPC + _WPC
            return (u // 2, u % 2)

        pltpu.emit_pipeline(
            body,
            grid=(n_win,),
            in_specs=[
                pl.BlockSpec((1, 64), index_map=amap),
                pl.BlockSpec((1, 64), index_map=bmap),
            ],
            out_specs=[
                pl.BlockSpec((_WINDOW, _DIM), index_map=lambda i: (i, 0)),
            ],
            core_axis_name=("c", "s"),
            dimension_semantics=(pltpu.PARALLEL,),
        )(i_hbm, i_hbm, o_hbm)

    return embed_gather(table_lin, idx_nat)


def _relayout_out(out_view, blk0, n_blk, prev):
    def body(x_ref, o_ref):
        y = jnp.swapaxes(x_ref[...], 0, 1)
        o_ref[...] = jnp.concatenate([y[0:64, :], y[64:128, :]], axis=1)
    operands = [out_view]
    in_specs = [pl.BlockSpec((_C_BLK // 2, 128), lambda r: (r, 0))]
    aliases = {}
    if prev is not None:
        operands.append(prev)
        in_specs.append(pl.BlockSpec(memory_space=pl.ANY))
        aliases = {1: 0}

    def body_wrap(*refs):
        body(refs[0], refs[-1])

    return pl.pallas_call(
        body_wrap,
        grid=(n_blk,),
        in_specs=in_specs,
        out_specs=pl.BlockSpec((64, _C_BLK), lambda r: (0, r + blk0)),
        out_shape=jax.ShapeDtypeStruct((64, _N_IDX), jnp.float32),
        input_output_aliases=aliases,
        compiler_params=pltpu.CompilerParams(
            dimension_semantics=("parallel",)
        ),
    )(*operands)


def kernel(token_id, emb_norm):
    idx_nat = token_id.reshape(_N_WINDOWS, _WINDOW).astype(jnp.int32)
    table_folded = _relayout_table(emb_norm.T)
    table_lin = table_folded.reshape(2 * _HALF, _DIM)
    out_t = None
    base = 0
    for n_win in _CHUNKS:
        gathered = _sc_gather(table_lin, idx_nat, base, n_win)
        out_t = _relayout_out(
            gathered.reshape(n_win * 64, 128),
            base // _WPC,
            n_win // _WPC,
            out_t,
        )
        base += n_win
    return out_t.T
